# baseline (speedup 1.0000x reference)
import functools

import jax
import jax.numpy as jnp
from jax import lax
from jax.experimental import pallas as pl
from jax.experimental.pallas import tpu as pltpu
from jax.experimental.pallas import tpu_sc as plsc

_KB = 4000


def _topk_body(nsteps, n_keys, d, q_ref, f_ref, sel_ref, maxv_ref, maxi_ref):
    j = pl.program_id(0)
    per_row = 128 // d
    planes = [
        lax.dot_general(
            q_ref[:], f_ref[:, d * p:d * (p + 1)],
            dimension_numbers=(((1,), (1,)), ((), ())),
            preferred_element_type=jnp.float32)
        for p in range(per_row)
    ]
    bmax = planes[0]
    for s in planes[1:]:
        bmax = jnp.maximum(bmax, s)
    bmax = jnp.max(bmax, axis=1, keepdims=True)
    cols = lax.broadcasted_iota(
        jnp.int32, planes[0].shape, 1).astype(jnp.float32)
    big = float(n_keys)
    bidx = None
    for p, s in enumerate(planes):
        m = jnp.min(jnp.where(s == bmax, cols, big),
                    axis=1, keepdims=True) * float(per_row) + float(p)
        bidx = m if bidx is None else jnp.minimum(bidx, m)
    bidx = (bidx + j * float(_KB)).astype(jnp.int32)

    @pl.when(j == 0)
    def _():
        maxv_ref[:] = bmax
        maxi_ref[:] = bidx

    @pl.when(j > 0)
    def _():
        upd = bmax > maxv_ref[:]
        maxi_ref[:] = jnp.where(upd, bidx, maxi_ref[:])
        maxv_ref[:] = jnp.where(upd, bmax, maxv_ref[:])

    @pl.when(j == nsteps - 1)
    def _():
        sel = sel_ref[:] != 0
        maxv_ref[:] = jnp.where(sel, maxv_ref[:], -1.0)


def _scan_topk(q, table128, sel_i32, n_keys, nsteps):
    qn, d = q.shape
    rows = _KB * d // 128
    body = functools.partial(_topk_body, nsteps, n_keys, d)
    return pl.pallas_call(
        body,
        grid=(nsteps,),
        in_specs=[
            pl.BlockSpec((qn, d), lambda j: (0, 0)),
            pl.BlockSpec((rows, 128), lambda j: (j, 0)),
            pl.BlockSpec((qn, 1), lambda j: (0, 0)),
        ],
        out_specs=[
            pl.BlockSpec((qn, 1), lambda j: (0, 0)),
            pl.BlockSpec((qn, 1), lambda j: (0, 0)),
        ],
        out_shape=[
            jax.ShapeDtypeStruct((qn, 1), jnp.float32),
            jax.ShapeDtypeStruct((qn, 1), jnp.int32),
        ],
        compiler_params=pltpu.CompilerParams(
            dimension_semantics=("arbitrary",)),
    )(q, table128, sel_i32)


def _sc_gather(table128, idx):
    b = idx.shape[0]
    info = plsc.get_sparse_core_info()
    nc = info.num_cores
    nw = nc * info.num_subcores
    b_per_w = b // nw
    mesh = plsc.VectorSubcoreMesh(core_axis_name="c", subcore_axis_name="s")

    @functools.partial(
        pl.kernel, mesh=mesh,
        out_type=jax.ShapeDtypeStruct((b, 128), jnp.float32),
        scratch_types=[
            pltpu.VMEM((b_per_w,), jnp.int32),
            pltpu.VMEM((b_per_w,), jnp.int32),
            pltpu.VMEM((b_per_w, 128), jnp.float32),
            pltpu.SemaphoreType.DMA,
        ],
    )
    def gather(table_hbm, idx_hbm, out_hbm, idx_v, grp_v, rows_v, sem):
        wid = lax.axis_index("s") * nc + lax.axis_index("c")
        base = wid * b_per_w
        pltpu.sync_copy(idx_hbm.at[pl.ds(base, b_per_w)], idx_v)
        for c in range(b_per_w // 16):
            v = idx_v[pl.ds(16 * c, 16)]
            grp_v[pl.ds(16 * c, 16)] = lax.shift_right_logical(v, 2)
        pltpu.async_copy(table_hbm.at[grp_v], rows_v, sem).wait()
        pltpu.sync_copy(rows_v, out_hbm.at[pl.ds(base, b_per_w)])

    return gather(table128, idx)


def _extract_body(d, per_row, wide_ref, idx_ref, sel_ref, out_ref):
    off = idx_ref[:] & (per_row - 1)
    acc = jnp.full(out_ref.shape, -1.0, jnp.float32)
    for t in range(per_row):
        take = jnp.logical_and(off == t, sel_ref[:] != 0)
        acc = jnp.where(take, wide_ref[:, d * t:d * (t + 1)], acc)
    out_ref[:] = acc


def _tc_extract(wide, idx2d, sel_i32, d):
    qn = wide.shape[0]
    per_row = 128 // d
    return pl.pallas_call(
        functools.partial(_extract_body, d, per_row),
        out_shape=jax.ShapeDtypeStruct((qn, d), jnp.float32),
    )(wide, idx2d, sel_i32)


def kernel(query_embeddings, selector, face_feats):
    qn, d = query_embeddings.shape
    n_keys = face_feats.shape[0]
    nsteps = n_keys // _KB
    sel_i32 = selector.astype(jnp.int32)[:, None]
    table128 = face_feats.reshape(-1, 128)
    maxv, idx2d = _scan_topk(query_embeddings, table128, sel_i32,
                             n_keys, nsteps)
    wide = _sc_gather(table128, idx2d.reshape(qn))
    target_embeddings = _tc_extract(wide, idx2d, sel_i32, d)
    similarities = maxv.reshape(qn)
    return (target_embeddings, similarities)


# speedup vs baseline: 1.0266x; 1.0266x over previous
import functools

import jax
import jax.numpy as jnp
from jax import lax
from jax.experimental import pallas as pl
from jax.experimental.pallas import tpu as pltpu
from jax.experimental.pallas import tpu_sc as plsc

_KB = 4000


def _topk_body(nsteps, n_keys, d, q_ref, f_ref, sel_ref, maxv_ref, maxi_ref,
               offrep_ref):
    j = pl.program_id(0)
    per_row = 128 // d
    planes = [
        lax.dot_general(
            q_ref[:], f_ref[:, d * p:d * (p + 1)],
            dimension_numbers=(((1,), (1,)), ((), ())),
            preferred_element_type=jnp.float32)
        for p in range(per_row)
    ]
    bmax = planes[0]
    for s in planes[1:]:
        bmax = jnp.maximum(bmax, s)
    bmax = jnp.max(bmax, axis=1, keepdims=True)
    cols = lax.broadcasted_iota(
        jnp.int32, planes[0].shape, 1).astype(jnp.float32) * float(per_row)
    big = float(n_keys)
    bidx = None
    for p, s in enumerate(planes):
        m = jnp.min(jnp.where(s == bmax, cols + float(p), big),
                    axis=1, keepdims=True)
        bidx = m if bidx is None else jnp.minimum(bidx, m)
    bidx = bidx + j * float(_KB)

    @pl.when(j == 0)
    def _():
        maxv_ref[:] = bmax
        maxi_ref[:] = bidx

    @pl.when(j > 0)
    def _():
        upd = bmax > maxv_ref[:]
        maxi_ref[:] = jnp.where(upd, bidx, maxi_ref[:])
        maxv_ref[:] = jnp.where(upd, bmax, maxv_ref[:])

    @pl.when(j == nsteps - 1)
    def _():
        sel = sel_ref[:] != 0
        maxv_ref[:] = jnp.where(sel, maxv_ref[:], -1.0)
        per = 128 // d
        off = jnp.where(sel, maxi_ref[:].astype(jnp.int32) & (per - 1), per)
        offrep_ref[:] = jnp.broadcast_to(off, offrep_ref.shape)


def _scan_topk(q, table128, sel_i32, n_keys, nsteps):
    qn, d = q.shape
    rows = _KB * d // 128
    body = functools.partial(_topk_body, nsteps, n_keys, d)
    return pl.pallas_call(
        body,
        grid=(nsteps,),
        in_specs=[
            pl.BlockSpec((qn, d), lambda j: (0, 0)),
            pl.BlockSpec((rows, 128), lambda j: (j, 0)),
            pl.BlockSpec((qn, 1), lambda j: (0, 0)),
        ],
        out_specs=[
            pl.BlockSpec((qn, 1), lambda j: (0, 0)),
            pl.BlockSpec((qn, 1), lambda j: (0, 0)),
            pl.BlockSpec((qn, 16), lambda j: (0, 0)),
        ],
        out_shape=[
            jax.ShapeDtypeStruct((qn, 1), jnp.float32),
            jax.ShapeDtypeStruct((qn, 1), jnp.float32),
            jax.ShapeDtypeStruct((qn, 16), jnp.int32),
        ],
        compiler_params=pltpu.CompilerParams(
            dimension_semantics=("arbitrary",)),
    )(q, table128, sel_i32)


def _sc_gather(table128, idx, offrep, d):
    b = idx.shape[0]
    per_row = 128 // d
    info = plsc.get_sparse_core_info()
    nc = info.num_cores
    nw = nc * info.num_subcores
    b_per_w = b // nw
    mesh = plsc.VectorSubcoreMesh(core_axis_name="c", subcore_axis_name="s")

    @functools.partial(
        pl.kernel, mesh=mesh,
        out_type=jax.ShapeDtypeStruct((b, d), jnp.float32),
        scratch_types=[
            pltpu.VMEM((b_per_w,), jnp.int32),
            pltpu.VMEM((b_per_w,), jnp.int32),
            pltpu.VMEM((b_per_w, 16), jnp.int32),
            pltpu.VMEM((b_per_w, 128), jnp.float32),
            pltpu.VMEM((b_per_w, d), jnp.float32),
            pltpu.SemaphoreType.DMA,
        ],
    )
    def gather(table_hbm, idx_hbm, off_hbm, out_hbm,
               idx_v, grp_v, offs_v, rows_v, out_v, sem):
        wid = lax.axis_index("s") * nc + lax.axis_index("c")
        base = wid * b_per_w
        pltpu.sync_copy(idx_hbm.at[pl.ds(base, b_per_w)], idx_v)
        for c in range(b_per_w // 16):
            v = idx_v[pl.ds(16 * c, 16)]
            grp_v[pl.ds(16 * c, 16)] = lax.shift_right_logical(v, 2)
        copy = pltpu.async_copy(table_hbm.at[grp_v], rows_v, sem)
        pltpu.sync_copy(off_hbm.at[pl.ds(base, b_per_w)], offs_v)
        copy.wait()
        for i in range(b_per_w):
            off = offs_v[i, :]
            for h in range(d // 16):
                acc = jnp.full((16,), -1.0, jnp.float32)
                for t in range(per_row):
                    acc = jnp.where(off == t,
                                    rows_v[i, pl.ds(d * t + 16 * h, 16)], acc)
                out_v[i, pl.ds(16 * h, 16)] = acc
        pltpu.sync_copy(out_v, out_hbm.at[pl.ds(base, b_per_w)])

    return gather(table128, idx, offrep)


def kernel(query_embeddings, selector, face_feats):
    qn, d = query_embeddings.shape
    n_keys = face_feats.shape[0]
    nsteps = n_keys // _KB
    sel_i32 = selector.astype(jnp.int32)[:, None]
    table128 = face_feats.reshape(-1, 128)
    maxv, maxi_f, offrep = _scan_topk(query_embeddings, table128, sel_i32,
                                      n_keys, nsteps)
    idx = maxi_f.astype(jnp.int32).reshape(qn)
    target_embeddings = _sc_gather(table128, idx, offrep, d)
    similarities = maxv.reshape(qn)
    return (target_embeddings, similarities)
